# Initial kernel scaffold; baseline (speedup 1.0000x reference)
import jax
import jax.numpy as jnp
from jax.experimental import pallas as pl


def kernel(tgt_token_ids_batch):
    raise NotImplementedError("write your pallas kernel here")


# speedup vs baseline: 1.8415x; 1.8415x over previous
import functools

import jax
import jax.numpy as jnp
from jax import lax
from jax.experimental import pallas as pl

BATCH = 1024
VOCAB = 100000
PADDING_IDX = 0

BLOCK_ROWS = 256
BLOCK_COLS = 4096


def _onehot_block(tgt_ref, out_ref):
    j = pl.program_id(1)
    ids = tgt_ref[:, :]
    col = lax.broadcasted_iota(jnp.int32, (BLOCK_ROWS, BLOCK_COLS), 1)
    col = col + j * BLOCK_COLS
    hit = (col == ids) & (ids != PADDING_IDX)
    out_ref[:, :] = hit.astype(jnp.float32)


@jax.jit
def kernel(tgt_token_ids_batch):
    tgt = tgt_token_ids_batch.astype(jnp.int32)
    grid = (BATCH // BLOCK_ROWS, pl.cdiv(VOCAB, BLOCK_COLS))
    return pl.pallas_call(
        _onehot_block,
        grid=grid,
        in_specs=[pl.BlockSpec((BLOCK_ROWS, 1), lambda i, j: (i, 0))],
        out_specs=pl.BlockSpec((BLOCK_ROWS, BLOCK_COLS), lambda i, j: (i, j)),
        out_shape=jax.ShapeDtypeStruct((BATCH, VOCAB), jnp.float32),
    )(tgt)


# speedup vs baseline: 1.8690x; 1.0149x over previous
import jax
import jax.numpy as jnp
from jax import lax
from jax.experimental import pallas as pl
from jax.experimental.pallas import tpu as pltpu

BATCH = 1024
VOCAB = 100000
PADDING_IDX = 0

CHUNK_ROWS = 32
SUB_ROWS = 8
NSUB = CHUNK_ROWS // SUB_ROWS
NBUF = 2
NCHUNK = BATCH // CHUNK_ROWS


def _onehot_chunk(tgt_ref, out_ref, buf0, buf1, sems):
    i = pl.program_id(0)
    ids = tgt_ref[:, :]
    base = i * CHUNK_ROWS

    def run(k, buf):
        @pl.when(i >= NBUF)
        def _wait_prev():
            for j in range(NSUB):
                pltpu.make_async_copy(
                    buf.at[pl.ds(j * SUB_ROWS, SUB_ROWS), :],
                    out_ref.at[pl.ds(base + j * SUB_ROWS, SUB_ROWS), :],
                    sems.at[k, j],
                ).wait()

        col = lax.broadcasted_iota(jnp.int32, (CHUNK_ROWS, VOCAB), 1)
        hit = (col == ids) & (ids != PADDING_IDX)
        buf[:, :] = hit.astype(jnp.float32)
        for j in range(NSUB):
            pltpu.make_async_copy(
                buf.at[pl.ds(j * SUB_ROWS, SUB_ROWS), :],
                out_ref.at[pl.ds(base + j * SUB_ROWS, SUB_ROWS), :],
                sems.at[k, j],
            ).start()

    lax.cond(i % NBUF == 0, lambda: run(0, buf0), lambda: run(1, buf1))

    @pl.when(i == NCHUNK - 1)
    def _drain():
        for k, buf in ((0, buf0), (1, buf1)):
            for j in range(NSUB):
                pltpu.make_async_copy(
                    buf.at[pl.ds(j * SUB_ROWS, SUB_ROWS), :],
                    out_ref.at[pl.ds(j * SUB_ROWS, SUB_ROWS), :],
                    sems.at[k, j],
                ).wait()


@jax.jit
def kernel(tgt_token_ids_batch):
    tgt = tgt_token_ids_batch.astype(jnp.int32)
    return pl.pallas_call(
        _onehot_chunk,
        grid=(NCHUNK,),
        in_specs=[pl.BlockSpec((CHUNK_ROWS, 1), lambda i: (i, 0))],
        out_specs=pl.BlockSpec(memory_space=pltpu.MemorySpace.HBM),
        out_shape=jax.ShapeDtypeStruct((BATCH, VOCAB), jnp.float32),
        scratch_shapes=[
            pltpu.VMEM((CHUNK_ROWS, VOCAB), jnp.float32),
            pltpu.VMEM((CHUNK_ROWS, VOCAB), jnp.float32),
            pltpu.SemaphoreType.DMA((NBUF, NSUB)),
        ],
        compiler_params=pltpu.CompilerParams(
            dimension_semantics=("arbitrary",),
        ),
    )(tgt)
